# baseline (speedup 1.0000x reference)
import functools

import jax
import jax.numpy as jnp
from jax import lax
from jax.experimental import pallas as pl
from jax.experimental.pallas import tpu as pltpu
from jax.experimental.pallas import tpu_sc as plsc

EMBED_DIM = 64
SEQ = 200
LANES = 16

NUM_CORES = 2
NUM_SUBCORES = 16
NUM_WORKERS = NUM_CORES * NUM_SUBCORES

CHUNK = 128
NBUF = 6
PE_REP = 2


def _pe_table():
    pos = jnp.arange(SEQ, dtype=jnp.float32)[:, None]
    i = jnp.arange(0, EMBED_DIM, 2, dtype=jnp.float32)
    div = jnp.exp(-(jnp.log(10000.0) * i / EMBED_DIM))
    pe = jnp.zeros((SEQ, EMBED_DIM), dtype=jnp.float32)
    pe = pe.at[:, 0::2].set(jnp.sin(pos * div))
    pe = pe.at[:, 1::2].set(jnp.cos(pos * div))
    return pe


def _make_kernel(batch, seq):
    total = batch * seq
    per_w = total // NUM_WORKERS
    steps = per_w // CHUNK
    assert per_w % CHUNK == 0 and steps > NBUF + 2
    assert (steps - NBUF - 2) % NBUF == 0

    mesh = plsc.VectorSubcoreMesh(
        core_axis_name="c", subcore_axis_name="s",
        num_cores=NUM_CORES, num_subcores=NUM_SUBCORES)

    @functools.partial(
        pl.kernel,
        out_type=jax.ShapeDtypeStruct((total, EMBED_DIM), jnp.float32),
        mesh=mesh,
        compiler_params=pltpu.CompilerParams(use_tc_tiling_on_sc=False),
        scratch_types=[
            pltpu.VMEM((steps, CHUNK), jnp.int32),
            pltpu.VMEM((PE_REP * SEQ, EMBED_DIM), jnp.float32),
            pltpu.VMEM((NBUF, CHUNK, EMBED_DIM), jnp.float32),
        ]
        + [pltpu.SemaphoreType.DMA] * (2 * NBUF),
    )
    def k(x_hbm, table_hbm, pe_hbm, out_hbm, idx_v, pe_v, rows_v, *sems):
        sem_g = sems[:NBUF]
        sem_s = sems[NBUF:]
        wid = lax.axis_index("s") * NUM_CORES + lax.axis_index("c")
        base = wid * per_w
        pltpu.sync_copy(x_hbm.at[wid], idx_v)
        pltpu.sync_copy(pe_hbm, pe_v)

        def start_gather(b, kstep):
            pltpu.async_copy(table_hbm.at[idx_v.at[kstep]], rows_v.at[b],
                             sem_g[b])

        def wait_gather(b):
            pltpu.make_async_copy(
                table_hbm.at[pl.ds(0, CHUNK)], rows_v.at[b], sem_g[b]).wait()

        def start_scatter(b, kstep):
            pltpu.async_copy(
                rows_v.at[b], out_hbm.at[pl.ds(base + kstep * CHUNK, CHUNK)],
                sem_s[b])

        def wait_scatter(b):
            pltpu.make_async_copy(
                rows_v.at[b], out_hbm.at[pl.ds(0, CHUNK)], sem_s[b]).wait()

        def add_pe(b, kstep):
            p0 = lax.rem(kstep * CHUNK, seq)

            def add_row(r, _):
                for c in range(EMBED_DIM // LANES):
                    sl = pl.ds(c * LANES, LANES)
                    plsc.addupdate(rows_v.at[b, r, sl], pe_v[p0 + r, sl])
                return 0

            lax.fori_loop(0, CHUNK, add_row, 0, unroll=8)

        def body(b, kstep, relaunch, scatter_wait):
            wait_gather(b)
            add_pe(b, kstep)
            start_scatter(b, kstep)
            if relaunch:
                bn = (b + 2) % NBUF
                if scatter_wait:
                    wait_scatter(bn)
                start_gather(bn, kstep + 2)

        start_gather(0, 0)
        start_gather(1, 1)

        for ks in range(NBUF):
            body(ks, ks, relaunch=True, scatter_wait=ks >= 4)

        groups = (steps - NBUF - 2) // NBUF

        def loop_body(g, _):
            k0s = NBUF + g * NBUF
            for off in range(NBUF):
                body(off, k0s + off, relaunch=True, scatter_wait=True)
            return 0

        lax.fori_loop(0, groups, loop_body, 0)

        body((steps - 2) % NBUF, steps - 2, relaunch=False, scatter_wait=False)
        body((steps - 1) % NBUF, steps - 1, relaunch=False, scatter_wait=False)

        for b in range(NBUF):
            wait_scatter(b)

    return k


def kernel(x, token_embedding_weight):
    batch, seq = x.shape
    total = batch * seq
    xi = x.astype(jnp.int32).reshape(NUM_WORKERS, total // (NUM_WORKERS * CHUNK), CHUNK)
    pe1 = _pe_table()
    pe = jnp.concatenate([pe1] * PE_REP, axis=0)
    k = _make_kernel(batch, seq)
    out = k(xi, token_embedding_weight, pe)
    return out.reshape(batch, seq, EMBED_DIM)
